# baseline (speedup 1.0000x reference)
import functools

import jax
import jax.numpy as jnp
from jax import lax
from jax.experimental import pallas as pl
from jax.experimental.pallas import tpu as pltpu
from jax.experimental.pallas import tpu_sc as plsc

BATCH = 16384
EMBED_DIM = 32
NUM_CORES = 2
NUM_SUBCORES = 16
LANES = 16
NUM_WORKERS = NUM_CORES * NUM_SUBCORES
B_PER_W = BATCH // NUM_WORKERS
CHUNK = 256
ROWS_PER_ENTRY = 4


def _body(uid_hbm, iid_hbm, ut_hbm, it_hbm, out_hbm,
          uid_v, iid_v, uent_v, ient_v, ubuf, ibuf, out_v, sem_u, sem_i):
    wid = lax.axis_index("s") * NUM_CORES + lax.axis_index("c")
    base = wid * B_PER_W

    pltpu.sync_copy(uid_hbm.at[pl.ds(base, B_PER_W)], uid_v)
    pltpu.sync_copy(iid_hbm.at[pl.ds(base, B_PER_W)], iid_v)

    def split(g, carry):
        c0 = g * LANES
        u = uid_v[pl.ds(c0, LANES)]
        i = iid_v[pl.ds(c0, LANES)]
        uent_v[pl.ds(c0, LANES)] = jax.lax.shift_right_logical(u, 2)
        ient_v[pl.ds(c0, LANES)] = jax.lax.shift_right_logical(i, 2)
        return carry
    lax.fori_loop(0, B_PER_W // LANES, split, 0)

    lane = lax.iota(jnp.int32, LANES)

    def chunk(h, carry):
        c0 = h * CHUNK
        cp_u = pltpu.async_copy(ut_hbm.at[uent_v.at[pl.ds(c0, CHUNK)]],
                                ubuf, sem_u)
        cp_i = pltpu.async_copy(it_hbm.at[ient_v.at[pl.ds(c0, CHUNK)]],
                                ibuf, sem_i)
        cp_u.wait()
        cp_i.wait()

        def group(g, carry2):
            gc = g * LANES
            cv = gc + lane
            uq = jax.lax.rem(uid_v[pl.ds(c0 + gc, LANES)], 4) * EMBED_DIM
            iq = jax.lax.rem(iid_v[pl.ds(c0 + gc, LANES)], 4) * EMBED_DIM
            acc = jnp.zeros((LANES,), jnp.float32)
            for j in range(EMBED_DIM):
                u = plsc.load_gather(ubuf, [cv, uq + j])
                v = plsc.load_gather(ibuf, [cv, iq + j])
                acc = acc + u * v
            out_v[pl.ds(c0 + gc, LANES)] = acc
            return carry2
        lax.fori_loop(0, CHUNK // LANES, group, 0)
        return carry
    lax.fori_loop(0, B_PER_W // CHUNK, chunk, 0)

    pltpu.sync_copy(out_v, out_hbm.at[pl.ds(base, B_PER_W)])


@jax.jit
def kernel(user_ids, item_ids, user_table, item_table):
    mesh = plsc.VectorSubcoreMesh(
        core_axis_name="c", subcore_axis_name="s",
        num_cores=NUM_CORES, num_subcores=NUM_SUBCORES)
    f = pl.kernel(
        _body,
        out_type=jax.ShapeDtypeStruct((BATCH,), jnp.float32),
        mesh=mesh,
        compiler_params=pltpu.CompilerParams(needs_layout_passes=False),
        scratch_types=[
            pltpu.VMEM((B_PER_W,), jnp.int32),
            pltpu.VMEM((B_PER_W,), jnp.int32),
            pltpu.VMEM((B_PER_W,), jnp.int32),
            pltpu.VMEM((B_PER_W,), jnp.int32),
            pltpu.VMEM((CHUNK, ROWS_PER_ENTRY * EMBED_DIM), jnp.float32),
            pltpu.VMEM((CHUNK, ROWS_PER_ENTRY * EMBED_DIM), jnp.float32),
            pltpu.VMEM((B_PER_W,), jnp.float32),
            pltpu.SemaphoreType.DMA,
            pltpu.SemaphoreType.DMA,
        ],
    )
    n_entries = user_table.shape[0] // ROWS_PER_ENTRY
    return f(user_ids.astype(jnp.int32), item_ids.astype(jnp.int32),
             user_table.reshape(n_entries, ROWS_PER_ENTRY * EMBED_DIM),
             item_table.reshape(n_entries, ROWS_PER_ENTRY * EMBED_DIM))
